# baseline (speedup 1.0000x reference)
import functools

import jax
import jax.numpy as jnp
from jax import lax
from jax.experimental import pallas as pl
from jax.experimental.pallas import tpu as pltpu
from jax.experimental.pallas import tpu_sc as plsc

D = 768
L = 16
NCOL = D // L
NC, NS = 2, 16
NW = NC * NS


def _fast_rsqrt(v):
    i = lax.bitcast_convert_type(v, jnp.int32)
    i = jnp.int32(0x5F3759DF) - lax.shift_right_logical(i, 1)
    y = lax.bitcast_convert_type(i, jnp.float32)
    for _ in range(3):
        y = y * (1.5 - 0.5 * v * y * y)
    return y


def _lane_sum(x):
    lanes = lax.iota(jnp.int32, L)
    for k in (1, 2, 4, 8):
        idx = lax.bitwise_xor(lanes, jnp.int32(k))
        x = x + lax.gather(
            x, idx[:, None],
            lax.GatherDimensionNumbers(
                offset_dims=(), collapsed_slice_dims=(0,),
                start_index_map=(0,)),
            slice_sizes=(1,),
            mode=lax.GatherScatterMode.PROMISE_IN_BOUNDS)
    return x


def _make_kernel(tokens, max_seq):
    per_w = tokens // NW
    g = 32
    nch = per_w // g

    mesh = plsc.VectorSubcoreMesh(core_axis_name="c", subcore_axis_name="s")

    @functools.partial(
        pl.kernel,
        mesh=mesh,
        out_type=jax.ShapeDtypeStruct((tokens, D), jnp.float32),
        scratch_types=[
            pltpu.VMEM((g,), jnp.int32),
            pltpu.VMEM((g,), jnp.int32),
            pltpu.VMEM((g, D), jnp.float32),
            pltpu.VMEM((g, D), jnp.float32),
            pltpu.VMEM((g, D), jnp.float32),
            pltpu.VMEM((D,), jnp.float32),
            pltpu.VMEM((D,), jnp.float32),
            pltpu.SemaphoreType.DMA,
            pltpu.SemaphoreType.DMA,
        ],
    )
    def bert_embed(ids_hbm, tt_hbm, tok_table, pos_table, seg_table,
                   gamma_hbm, beta_hbm, out_hbm,
                   idx_v, tti_v, x_v, pos_v, seg_v, g_v, b_v, sem, sem2):
        wid = lax.axis_index("s") * NC + lax.axis_index("c")
        pltpu.sync_copy(gamma_hbm, g_v)
        pltpu.sync_copy(beta_hbm, b_v)

        def chunk_body(i, carry):
            base = wid * per_w + i * g
            s0 = lax.rem(base, max_seq)
            pltpu.sync_copy(ids_hbm.at[pl.ds(base, g)], idx_v)
            pltpu.sync_copy(tt_hbm.at[pl.ds(base, g)], tti_v)
            cp1 = pltpu.async_copy(tok_table.at[idx_v], x_v, sem)
            cp2 = pltpu.async_copy(seg_table.at[tti_v], seg_v, sem2)
            pltpu.sync_copy(pos_table.at[pl.ds(s0, g)], pos_v)
            cp1.wait()
            cp2.wait()

            def row_body(r, rcarry):
                s1 = jnp.zeros((L,), jnp.float32)
                s2 = jnp.zeros((L,), jnp.float32)
                for c in range(NCOL):
                    sl = pl.ds(c * L, L)
                    x = x_v[r, sl] + pos_v[r, sl] + seg_v[r, sl]
                    x_v[r, sl] = x
                    s1 = s1 + x
                    s2 = s2 + x * x
                mean = _lane_sum(s1) * (1.0 / D)
                var = _lane_sum(s2) * (1.0 / D) - mean * mean
                inv = _fast_rsqrt(var + 1e-12)
                for c in range(NCOL):
                    sl = pl.ds(c * L, L)
                    y = (x_v[r, sl] - mean) * inv
                    x_v[r, sl] = y * g_v[sl] + b_v[sl]
                return rcarry

            lax.fori_loop(0, g, row_body, 0)
            pltpu.sync_copy(x_v, out_hbm.at[pl.ds(base, g)])
            return carry

        lax.fori_loop(0, nch, chunk_body, 0)

    return bert_embed


def kernel(input_ids, token_type_ids, token_table, pos_table, seg_table,
           gamma, beta):
    batch, max_seq = input_ids.shape
    tokens = batch * max_seq
    ids = input_ids.reshape(tokens).astype(jnp.int32)
    tt = token_type_ids.reshape(tokens).astype(jnp.int32)
    out = _make_kernel(tokens, max_seq)(
        ids, tt, token_table, pos_table, seg_table, gamma, beta)
    return out.reshape(batch, max_seq, D)


# speedup vs baseline: 1.0910x; 1.0910x over previous
import functools

import jax
import jax.numpy as jnp
from jax import lax
from jax.experimental import pallas as pl
from jax.experimental.pallas import tpu as pltpu
from jax.experimental.pallas import tpu_sc as plsc

D = 768
L = 16
NCOL = D // L
NC, NS = 2, 16
NW = NC * NS


def _fast_rsqrt(v):
    i = lax.bitcast_convert_type(v, jnp.int32)
    i = jnp.int32(0x5F3759DF) - lax.shift_right_logical(i, 1)
    y = lax.bitcast_convert_type(i, jnp.float32)
    for _ in range(3):
        y = y * (1.5 - 0.5 * v * y * y)
    return y


def _lane_sum(x):
    lanes = lax.iota(jnp.int32, L)
    for k in (1, 2, 4, 8):
        idx = lax.bitwise_xor(lanes, jnp.int32(k))
        x = x + lax.gather(
            x, idx[:, None],
            lax.GatherDimensionNumbers(
                offset_dims=(), collapsed_slice_dims=(0,),
                start_index_map=(0,)),
            slice_sizes=(1,),
            mode=lax.GatherScatterMode.PROMISE_IN_BOUNDS)
    return x


def _make_kernel(tokens, max_seq):
    per_w = tokens // NW
    g = 16
    nch = per_w // g

    mesh = plsc.VectorSubcoreMesh(core_axis_name="c", subcore_axis_name="s")

    @functools.partial(
        pl.kernel,
        mesh=mesh,
        out_type=jax.ShapeDtypeStruct((tokens, D), jnp.float32),
        scratch_types=[
            pltpu.VMEM((per_w,), jnp.int32),
            pltpu.VMEM((per_w,), jnp.int32),
            pltpu.VMEM((2, g, D), jnp.float32),
            pltpu.VMEM((2, g, D), jnp.float32),
            pltpu.VMEM((2, g, D), jnp.float32),
            pltpu.VMEM((D,), jnp.float32),
            pltpu.VMEM((D,), jnp.float32),
            pltpu.SemaphoreType.DMA((2,)),
            pltpu.SemaphoreType.DMA,
        ],
    )
    def bert_embed(ids_hbm, tt_hbm, tok_table, pos_table, seg_table,
                   gamma_hbm, beta_hbm, out_hbm,
                   idx_v, tti_v, x_v, pos_v, seg_v, g_v, b_v,
                   in_sem, out_sem):
        wid = lax.axis_index("s") * NC + lax.axis_index("c")
        w0 = wid * per_w
        pltpu.sync_copy(gamma_hbm, g_v)
        pltpu.sync_copy(beta_hbm, b_v)
        pltpu.sync_copy(ids_hbm.at[pl.ds(w0, per_w)], idx_v)
        pltpu.sync_copy(tt_hbm.at[pl.ds(w0, per_w)], tti_v)

        def issue(i, buf):
            off = i * g
            s0 = lax.rem(w0 + off, max_seq)
            pltpu.async_copy(tok_table.at[idx_v.at[pl.ds(off, g)]],
                             x_v.at[buf], in_sem.at[buf])
            pltpu.async_copy(seg_table.at[tti_v.at[pl.ds(off, g)]],
                             seg_v.at[buf], in_sem.at[buf])
            pltpu.async_copy(pos_table.at[pl.ds(s0, g)],
                             pos_v.at[buf], in_sem.at[buf])

        def wait_in(buf):
            src = tok_table.at[pl.ds(0, g)]
            pltpu.make_async_copy(src, x_v.at[buf], in_sem.at[buf]).wait()
            pltpu.make_async_copy(src, seg_v.at[buf], in_sem.at[buf]).wait()
            pltpu.make_async_copy(src, pos_v.at[buf], in_sem.at[buf]).wait()

        def wait_out(buf):
            pltpu.make_async_copy(x_v.at[buf], out_hbm.at[pl.ds(0, g)],
                                  out_sem).wait()

        issue(0, 0)

        def chunk_body(i, carry):
            p = lax.rem(i, 2)
            q = 1 - p

            @pl.when(i > 0)
            def _():
                wait_out(q)

            @pl.when(i < nch - 1)
            def _():
                issue(i + 1, q)

            wait_in(p)

            def row_body(r, rcarry):
                s1 = jnp.zeros((L,), jnp.float32)
                s2 = jnp.zeros((L,), jnp.float32)
                for c in range(NCOL):
                    sl = pl.ds(c * L, L)
                    x = x_v[p, r, sl] + pos_v[p, r, sl] + seg_v[p, r, sl]
                    x_v[p, r, sl] = x
                    s1 = s1 + x
                    s2 = s2 + x * x
                mean = _lane_sum(s1) * (1.0 / D)
                var = _lane_sum(s2) * (1.0 / D) - mean * mean
                inv = _fast_rsqrt(var + 1e-12)
                for c in range(NCOL):
                    sl = pl.ds(c * L, L)
                    y = (x_v[p, r, sl] - mean) * inv
                    x_v[p, r, sl] = y * g_v[sl] + b_v[sl]
                return rcarry

            lax.fori_loop(0, g, row_body, 0)
            pltpu.async_copy(x_v.at[p], out_hbm.at[pl.ds(w0 + i * g, g)],
                             out_sem)
            return carry

        lax.fori_loop(0, nch, chunk_body, 0)
        wait_out(1)

    return bert_embed


def kernel(input_ids, token_type_ids, token_table, pos_table, seg_table,
           gamma, beta):
    batch, max_seq = input_ids.shape
    tokens = batch * max_seq
    ids = input_ids.reshape(tokens).astype(jnp.int32)
    tt = token_type_ids.reshape(tokens).astype(jnp.int32)
    out = _make_kernel(tokens, max_seq)(
        ids, tt, token_table, pos_table, seg_table, gamma, beta)
    return out.reshape(batch, max_seq, D)


# speedup vs baseline: 3.9876x; 3.6549x over previous
import functools

import jax
import jax.numpy as jnp
from jax import lax
from jax.experimental import pallas as pl
from jax.experimental.pallas import tpu as pltpu
from jax.experimental.pallas import tpu_sc as plsc

D = 768
NC, NS = 2, 16
NW = NC * NS


def _make_sc_gather(tokens):
    per_w = tokens // NW
    g = 64
    nch = per_w // g

    mesh = plsc.VectorSubcoreMesh(core_axis_name="c", subcore_axis_name="s")

    @functools.partial(
        pl.kernel,
        mesh=mesh,
        out_type=jax.ShapeDtypeStruct((tokens, D), jnp.float32),
        scratch_types=[
            pltpu.VMEM((per_w,), jnp.int32),
            pltpu.VMEM((2, g, D), jnp.float32),
            pltpu.SemaphoreType.DMA((2,)),
            pltpu.SemaphoreType.DMA,
        ],
    )
    def sc_gather(ids_hbm, table, out_hbm, idx_v, x_v, in_sem, out_sem):
        wid = lax.axis_index("s") * NC + lax.axis_index("c")
        w0 = wid * per_w
        pltpu.sync_copy(ids_hbm.at[pl.ds(w0, per_w)], idx_v)

        def issue(i, buf):
            pltpu.async_copy(table.at[idx_v.at[pl.ds(i * g, g)]],
                             x_v.at[buf], in_sem.at[buf])

        issue(0, 0)

        def chunk_body(i, carry):
            p = lax.rem(i, 2)
            q = 1 - p

            @pl.when(i > 0)
            def _():
                pltpu.make_async_copy(x_v.at[q], out_hbm.at[pl.ds(0, g)],
                                      out_sem).wait()

            @pl.when(i < nch - 1)
            def _():
                issue(i + 1, q)

            pltpu.make_async_copy(table.at[pl.ds(0, g)], x_v.at[p],
                                  in_sem.at[p]).wait()
            pltpu.async_copy(x_v.at[p], out_hbm.at[pl.ds(w0 + i * g, g)],
                             out_sem)
            return carry

        lax.fori_loop(0, nch, chunk_body, 0)
        pltpu.make_async_copy(x_v.at[(nch - 1) % 2], out_hbm.at[pl.ds(0, g)],
                              out_sem).wait()

    return sc_gather


def _tc_ln_body(tok_ref, pos_ref, ttf_ref, seg_ref, g_ref, b_ref, o_ref):
    seg0 = seg_ref[0:1, :]
    segd = seg_ref[1:2, :] - seg0
    x = tok_ref[...] + pos_ref[...] + (seg0 + ttf_ref[...] * segd)
    mean = jnp.mean(x, axis=1, keepdims=True)
    xc = x - mean
    var = jnp.mean(xc * xc, axis=1, keepdims=True)
    inv = lax.rsqrt(var + 1e-12)
    o_ref[...] = xc * inv * g_ref[...] + b_ref[...]


def _tc_layernorm(tok_rows, pos_table, ttf, seg_table, gamma, beta):
    tokens = tok_rows.shape[0]
    max_seq = pos_table.shape[0]
    blk = 512
    nblk = tokens // blk
    pos_per = max_seq // blk
    return pl.pallas_call(
        _tc_ln_body,
        grid=(nblk,),
        in_specs=[
            pl.BlockSpec((blk, D), lambda i: (i, 0)),
            pl.BlockSpec((blk, D), lambda i: (lax.rem(i, pos_per), 0)),
            pl.BlockSpec((blk, 1), lambda i: (i, 0)),
            pl.BlockSpec((2, D), lambda i: (0, 0)),
            pl.BlockSpec((1, D), lambda i: (0, 0)),
            pl.BlockSpec((1, D), lambda i: (0, 0)),
        ],
        out_specs=pl.BlockSpec((blk, D), lambda i: (i, 0)),
        out_shape=jax.ShapeDtypeStruct((tokens, D), jnp.float32),
    )(tok_rows, pos_table, ttf, seg_table, gamma, beta)


def kernel(input_ids, token_type_ids, token_table, pos_table, seg_table,
           gamma, beta):
    batch, max_seq = input_ids.shape
    tokens = batch * max_seq
    ids = input_ids.reshape(tokens).astype(jnp.int32)
    ttf = token_type_ids.reshape(tokens, 1).astype(jnp.float32)
    tok_rows = _make_sc_gather(tokens)(ids, token_table)
    out = _tc_layernorm(tok_rows, pos_table, ttf, seg_table,
                        gamma.reshape(1, D), beta.reshape(1, D))
    return out.reshape(batch, max_seq, D)


# speedup vs baseline: 4.1345x; 1.0368x over previous
import functools

import jax
import jax.numpy as jnp
from jax import lax
from jax.experimental import pallas as pl
from jax.experimental.pallas import tpu as pltpu
from jax.experimental.pallas import tpu_sc as plsc

D = 768
NC, NS = 2, 16
NW = NC * NS


def _make_sc_gather(tokens):
    per_w = tokens // NW
    g = 64
    nch = per_w // g

    mesh = plsc.VectorSubcoreMesh(core_axis_name="c", subcore_axis_name="s")

    @functools.partial(
        pl.kernel,
        mesh=mesh,
        out_type=jax.ShapeDtypeStruct((tokens, D), jnp.float32),
        scratch_types=[
            pltpu.VMEM((per_w,), jnp.int32),
            pltpu.VMEM((2, g, D), jnp.float32),
            pltpu.SemaphoreType.DMA((2,)),
            pltpu.SemaphoreType.DMA,
        ],
    )
    def sc_gather(ids_hbm, table, out_hbm, idx_v, x_v, in_sem, out_sem):
        wid = lax.axis_index("s") * NC + lax.axis_index("c")
        w0 = wid * per_w
        pltpu.sync_copy(ids_hbm.at[pl.ds(w0, per_w)], idx_v)

        def issue(i, buf):
            pltpu.async_copy(table.at[idx_v.at[pl.ds(i * g, g)]],
                             x_v.at[buf], in_sem.at[buf])

        issue(0, 0)

        def chunk_body(i, carry):
            p = lax.rem(i, 2)
            q = 1 - p

            @pl.when(i > 0)
            def _():
                pltpu.make_async_copy(x_v.at[q], out_hbm.at[pl.ds(0, g)],
                                      out_sem).wait()

            @pl.when(i < nch - 1)
            def _():
                issue(i + 1, q)

            pltpu.make_async_copy(table.at[pl.ds(0, g)], x_v.at[p],
                                  in_sem.at[p]).wait()
            pltpu.async_copy(x_v.at[p], out_hbm.at[pl.ds(w0 + i * g, g)],
                             out_sem)
            return carry

        lax.fori_loop(0, nch, chunk_body, 0)
        pltpu.make_async_copy(x_v.at[(nch - 1) % 2], out_hbm.at[pl.ds(0, g)],
                              out_sem).wait()

    return sc_gather


def _tc_ln_body(tok_ref, pos_ref, ttf_ref, seg_ref, g_ref, b_ref, o_ref):
    seg0 = seg_ref[0:1, :]
    segd = seg_ref[1:2, :] - seg0
    x = tok_ref[...] + pos_ref[...] + (seg0 + ttf_ref[...] * segd)
    mean = jnp.mean(x, axis=1, keepdims=True)
    xc = x - mean
    var = jnp.mean(xc * xc, axis=1, keepdims=True)
    inv = lax.rsqrt(var + 1e-12)
    o_ref[...] = xc * inv * g_ref[...] + b_ref[...]


def _tc_layernorm(tok_rows, pos_table, ttf, seg_table, gamma, beta):
    tokens = tok_rows.shape[0]
    max_seq = pos_table.shape[0]
    blk = 512
    pos_per = max_seq // blk
    batch = tokens // max_seq
    return pl.pallas_call(
        _tc_ln_body,
        grid=(pos_per, batch),
        in_specs=[
            pl.BlockSpec((blk, D), lambda j, b: (b * pos_per + j, 0)),
            pl.BlockSpec((blk, D), lambda j, b: (j, 0)),
            pl.BlockSpec((blk, 1), lambda j, b: (b * pos_per + j, 0)),
            pl.BlockSpec((2, D), lambda j, b: (0, 0)),
            pl.BlockSpec((1, D), lambda j, b: (0, 0)),
            pl.BlockSpec((1, D), lambda j, b: (0, 0)),
        ],
        out_specs=pl.BlockSpec((blk, D), lambda j, b: (b * pos_per + j, 0)),
        out_shape=jax.ShapeDtypeStruct((tokens, D), jnp.float32),
    )(tok_rows, pos_table, ttf, seg_table, gamma, beta)


def kernel(input_ids, token_type_ids, token_table, pos_table, seg_table,
           gamma, beta):
    batch, max_seq = input_ids.shape
    tokens = batch * max_seq
    ids = input_ids.reshape(tokens).astype(jnp.int32)
    ttf = token_type_ids.reshape(tokens, 1).astype(jnp.float32)
    tok_rows = _make_sc_gather(tokens)(ids, token_table)
    out = _tc_layernorm(tok_rows, pos_table, ttf, seg_table,
                        gamma.reshape(1, D), beta.reshape(1, D))
    return out.reshape(batch, max_seq, D)


# speedup vs baseline: 4.1942x; 1.0144x over previous
import functools

import jax
import jax.numpy as jnp
from jax import lax
from jax.experimental import pallas as pl
from jax.experimental.pallas import tpu as pltpu
from jax.experimental.pallas import tpu_sc as plsc

D = 768
NC, NS = 2, 16
NW = NC * NS


def _make_sc_gather(tokens):
    per_w = tokens // NW
    g = 32
    nb = 4
    nch = per_w // g

    mesh = plsc.VectorSubcoreMesh(core_axis_name="c", subcore_axis_name="s")

    @functools.partial(
        pl.kernel,
        mesh=mesh,
        out_type=jax.ShapeDtypeStruct((tokens, D), jnp.float32),
        scratch_types=[
            pltpu.VMEM((per_w,), jnp.int32),
            pltpu.VMEM((nb, g, D), jnp.float32),
            pltpu.SemaphoreType.DMA((nb,)),
            pltpu.SemaphoreType.DMA((nb,)),
        ],
    )
    def sc_gather(ids_hbm, table, out_hbm, idx_v, x_v, in_sem, out_sem):
        wid = lax.axis_index("s") * NC + lax.axis_index("c")
        w0 = wid * per_w
        pltpu.sync_copy(ids_hbm.at[pl.ds(w0, per_w)], idx_v)

        def gather(i, p):
            pltpu.async_copy(table.at[idx_v.at[pl.ds(i * g, g)]],
                             x_v.at[p], in_sem.at[p])

        for p in range(nb):
            gather(p, p)
        for i in range(nch):
            p = i % nb
            pltpu.make_async_copy(table.at[pl.ds(0, g)], x_v.at[p],
                                  in_sem.at[p]).wait()
            pltpu.async_copy(x_v.at[p], out_hbm.at[pl.ds(w0 + i * g, g)],
                             out_sem.at[p])
            if i + nb < nch:
                pltpu.make_async_copy(x_v.at[p], out_hbm.at[pl.ds(0, g)],
                                      out_sem.at[p]).wait()
                gather(i + nb, p)
        for i in range(nch - nb, nch):
            p = i % nb
            pltpu.make_async_copy(x_v.at[p], out_hbm.at[pl.ds(0, g)],
                                  out_sem.at[p]).wait()

    return sc_gather


def _tc_ln_body(tok_ref, pos_ref, ttf_ref, seg_ref, g_ref, b_ref, o_ref):
    seg0 = seg_ref[0:1, :]
    segd = seg_ref[1:2, :] - seg0
    x = tok_ref[...] + pos_ref[...] + (seg0 + ttf_ref[...] * segd)
    mean = jnp.mean(x, axis=1, keepdims=True)
    xc = x - mean
    var = jnp.mean(xc * xc, axis=1, keepdims=True)
    inv = lax.rsqrt(var + 1e-12)
    o_ref[...] = xc * inv * g_ref[...] + b_ref[...]


def _tc_layernorm(tok_rows, pos_table, ttf, seg_table, gamma, beta):
    tokens = tok_rows.shape[0]
    max_seq = pos_table.shape[0]
    blk = 512
    pos_per = max_seq // blk
    batch = tokens // max_seq
    return pl.pallas_call(
        _tc_ln_body,
        grid=(pos_per, batch),
        in_specs=[
            pl.BlockSpec((blk, D), lambda j, b: (b * pos_per + j, 0)),
            pl.BlockSpec((blk, D), lambda j, b: (j, 0)),
            pl.BlockSpec((blk, 1), lambda j, b: (b * pos_per + j, 0)),
            pl.BlockSpec((2, D), lambda j, b: (0, 0)),
            pl.BlockSpec((1, D), lambda j, b: (0, 0)),
            pl.BlockSpec((1, D), lambda j, b: (0, 0)),
        ],
        out_specs=pl.BlockSpec((blk, D), lambda j, b: (b * pos_per + j, 0)),
        out_shape=jax.ShapeDtypeStruct((tokens, D), jnp.float32),
    )(tok_rows, pos_table, ttf, seg_table, gamma, beta)


def kernel(input_ids, token_type_ids, token_table, pos_table, seg_table,
           gamma, beta):
    batch, max_seq = input_ids.shape
    tokens = batch * max_seq
    ids = input_ids.reshape(tokens).astype(jnp.int32)
    ttf = token_type_ids.reshape(tokens, 1).astype(jnp.float32)
    tok_rows = _make_sc_gather(tokens)(ids, token_table)
    out = _tc_layernorm(tok_rows, pos_table, ttf, seg_table,
                        gamma.reshape(1, D), beta.reshape(1, D))
    return out.reshape(batch, max_seq, D)
